# Initial kernel scaffold; baseline (speedup 1.0000x reference)
import jax
import jax.numpy as jnp
from jax.experimental import pallas as pl


def kernel(h, coord, edge_index, coord_diff, edge_attr, w1, b1, w2, b2, w3):
    raise NotImplementedError("write your pallas kernel here")


# speedup vs baseline: 2.2625x; 2.2625x over previous
import jax
import jax.numpy as jnp
from jax import lax
from jax.experimental import pallas as pl
from jax.experimental.pallas import tpu as pltpu
from jax.experimental.pallas import tpu_sc as plsc

N = 10000
E = 160000
H = 256
NC, NS, L = 2, 16, 16
NW = NC * NS
CHUNK = 128
NCHUNK = E // CHUNK
KMAX = (NCHUNK + NW - 1) // NW
N4 = N * 4
BE = 4000
NORM_INV = 1.0 / 100.0
CRANGE = 100.0


def _proj_body(h_ref, w_ref, out_ref):
    out_ref[0] = jnp.dot(h_ref[...], w_ref[0], preferred_element_type=jnp.float32)


def _project(h, w_stack):
    return pl.pallas_call(
        _proj_body,
        grid=(2,),
        in_specs=[
            pl.BlockSpec((N, H), lambda j: (0, 0)),
            pl.BlockSpec((1, H, H), lambda j: (j, 0, 0)),
        ],
        out_specs=pl.BlockSpec((1, N, H), lambda j: (j, 0, 0)),
        out_shape=jax.ShapeDtypeStruct((2, N, H), jnp.float32),
    )(h, w_stack)


def _gather_body(t_hbm, ridx_hbm, cidx_hbm, g_hbm,
                 ridx_v, cidx_v, buf_a, buf_b, sem_a, sem_b):
    wid = lax.axis_index("s") * NC + lax.axis_index("c")

    def chunk_body(k, carry):
        chunk = k * NW + wid

        @pl.when(chunk < NCHUNK)
        def _():
            base = chunk * CHUNK
            pltpu.sync_copy(ridx_hbm.at[pl.ds(base, CHUNK)], ridx_v)
            pltpu.sync_copy(cidx_hbm.at[pl.ds(base, CHUNK)], cidx_v)
            cp_a = pltpu.async_copy(t_hbm.at[ridx_v], buf_a, sem_a)
            cp_b = pltpu.async_copy(t_hbm.at[cidx_v], buf_b, sem_b)
            cp_a.wait()
            cp_b.wait()

            def add_body(e, c2):
                for j in range(H // L):
                    s = pl.ds(j * L, L)
                    buf_a[e, s] = buf_a[e, s] + buf_b[e, s]
                return c2

            lax.fori_loop(0, CHUNK, add_body, 0)
            pltpu.sync_copy(buf_a, g_hbm.at[pl.ds(base, CHUNK), :])

        return carry

    lax.fori_loop(0, KMAX, chunk_body, 0)


def _gather(t_table, ridx, cidx):
    mesh = plsc.VectorSubcoreMesh(
        core_axis_name="c", subcore_axis_name="s",
        num_cores=NC, num_subcores=NS)
    f = pl.kernel(
        _gather_body,
        out_type=jax.ShapeDtypeStruct((E, H), jnp.float32),
        mesh=mesh,
        scratch_types=[
            pltpu.VMEM((CHUNK,), jnp.int32),
            pltpu.VMEM((CHUNK,), jnp.int32),
            pltpu.VMEM((CHUNK, H), jnp.float32),
            pltpu.VMEM((CHUNK, H), jnp.float32),
            pltpu.SemaphoreType.DMA,
            pltpu.SemaphoreType.DMA,
        ],
    )
    return f(t_table, ridx, cidx)


def _mlp_body(g_ref, ea_ref, cd_ref, w1c_ref, b1_ref, w2_ref, b2_ref, w3_ref,
              out_ref):
    z = (g_ref[...]
         + jnp.dot(ea_ref[...], w1c_ref[...], preferred_element_type=jnp.float32)
         + b1_ref[...])
    x1 = z * jax.nn.sigmoid(z)
    y = jnp.dot(x1, w2_ref[...], preferred_element_type=jnp.float32) + b2_ref[...]
    x2 = y * jax.nn.sigmoid(y)
    m = jnp.sum(x2 * w3_ref[...], axis=1, keepdims=True)
    out_ref[...] = cd_ref[...] * (jnp.tanh(m) * CRANGE)


def _mlp(g, ea8, cd4, w1c8, b1r, w2t, b2r, w3r):
    grid = (E // BE,)
    return pl.pallas_call(
        _mlp_body,
        grid=grid,
        in_specs=[
            pl.BlockSpec((BE, H), lambda i: (i, 0)),
            pl.BlockSpec((BE, 8), lambda i: (i, 0)),
            pl.BlockSpec((BE, 4), lambda i: (i, 0)),
            pl.BlockSpec((8, H), lambda i: (0, 0)),
            pl.BlockSpec((1, H), lambda i: (0, 0)),
            pl.BlockSpec((H, H), lambda i: (0, 0)),
            pl.BlockSpec((1, H), lambda i: (0, 0)),
            pl.BlockSpec((1, H), lambda i: (0, 0)),
        ],
        out_specs=pl.BlockSpec((BE, 4), lambda i: (i, 0)),
        out_shape=jax.ShapeDtypeStruct((E, 4), jnp.float32),
    )(g, ea8, cd4, w1c8, b1r, w2t, b2r, w3r)


ZSEG = 1000


def _scatter_body(t_hbm, ridx_hbm, zeros_hbm, part_hbm, ridx_v, tval_v, shared):
    cid = lax.axis_index("c")
    sid = lax.axis_index("s")
    wid = sid * NC + cid

    @pl.when(sid < 10)
    def _():
        pltpu.sync_copy(zeros_hbm.at[pl.ds(sid * ZSEG, ZSEG), :],
                        shared.at[pl.ds(sid * ZSEG, ZSEG), :])

    plsc.subcore_barrier()

    def chunk_body(k, carry):
        chunk = k * NW + wid

        @pl.when(chunk < NCHUNK)
        def _():
            base = chunk * CHUNK
            pltpu.sync_copy(ridx_hbm.at[pl.ds(base, CHUNK)], ridx_v)
            pltpu.sync_copy(t_hbm.at[pl.ds(base, CHUNK), :], tval_v)
            pltpu.sync_copy(tval_v, shared.at[ridx_v], add=True)

        return carry

    lax.fori_loop(0, KMAX, chunk_body, 0)
    plsc.subcore_barrier()

    @pl.when(sid < 10)
    def _():
        pltpu.sync_copy(shared.at[pl.ds(sid * ZSEG, ZSEG), :],
                        part_hbm.at[cid, pl.ds(sid * ZSEG, ZSEG), :])


def _scatter(t4, ridx, zeros2d):
    mesh = plsc.VectorSubcoreMesh(
        core_axis_name="c", subcore_axis_name="s",
        num_cores=NC, num_subcores=NS)
    f = pl.kernel(
        _scatter_body,
        out_type=jax.ShapeDtypeStruct((NC, N, 4), jnp.float32),
        mesh=mesh,
        scratch_types=[
            pltpu.VMEM((CHUNK,), jnp.int32),
            pltpu.VMEM((CHUNK, 4), jnp.float32),
            pltpu.VMEM_SHARED((N, 4), jnp.float32),
        ],
        compiler_params=pltpu.CompilerParams(needs_layout_passes=False),
    )
    return f(t4, ridx, zeros2d)


def _final_body(p_ref, c_ref, out_ref):
    s = jnp.sum(p_ref[...], axis=0, keepdims=True)
    out_ref[...] = c_ref[...] + s * NORM_INV


def _final(part, coordp):
    return pl.pallas_call(
        _final_body,
        in_specs=[
            pl.BlockSpec((NC, N4), lambda: (0, 0)),
            pl.BlockSpec((1, N4), lambda: (0, 0)),
        ],
        out_specs=pl.BlockSpec((1, N4), lambda: (0, 0)),
        out_shape=jax.ShapeDtypeStruct((1, N4), jnp.float32),
    )(part, coordp)


def kernel(h, coord, edge_index, coord_diff, edge_attr, w1, b1, w2, b2, w3):
    row = edge_index[0].astype(jnp.int32)
    col = edge_index[1].astype(jnp.int32)

    w_stack = jnp.stack([w1[:, :H].T, w1[:, H:2 * H].T])
    t_table = _project(h, w_stack).reshape(2 * N, H)

    g = _gather(t_table, row, col + N)

    w1c8 = jnp.pad(w1[:, 2 * H:].T, ((0, 4), (0, 0)))
    ea8 = jnp.pad(edge_attr, ((0, 0), (0, 4)))
    cd4 = jnp.pad(coord_diff, ((0, 0), (0, 1)))
    t4 = _mlp(g, ea8, cd4, w1c8,
              b1.reshape(1, H), w2.T, b2.reshape(1, H), w3)

    zeros2d = jnp.zeros((N, 4), jnp.float32)
    part = _scatter(t4, row, zeros2d).reshape(NC, N4)

    coordp = jnp.pad(coord, ((0, 0), (0, 1))).reshape(1, N4)
    out = _final(part, coordp)
    return out.reshape(-1, 4)[:N, :3]


# speedup vs baseline: 2.2669x; 1.0019x over previous
import jax
import jax.numpy as jnp
from jax import lax
from jax.experimental import pallas as pl
from jax.experimental.pallas import tpu as pltpu
from jax.experimental.pallas import tpu_sc as plsc

N = 10000
E = 160000
H = 256
NC, NS, L = 2, 16, 16
NW = NC * NS
CHUNK = 128
NCHUNK = E // CHUNK
KMAX = (NCHUNK + NW - 1) // NW
N4 = N * 4
BE = 4000
NORM_INV = 1.0 / 100.0
CRANGE = 100.0


def _proj_body(h_ref, w_ref, out_ref):
    out_ref[0] = jnp.dot(h_ref[...], w_ref[0], preferred_element_type=jnp.float32)


def _project(h, w_stack):
    return pl.pallas_call(
        _proj_body,
        grid=(2,),
        in_specs=[
            pl.BlockSpec((N, H), lambda j: (0, 0)),
            pl.BlockSpec((1, H, H), lambda j: (j, 0, 0)),
        ],
        out_specs=pl.BlockSpec((1, N, H), lambda j: (j, 0, 0)),
        out_shape=jax.ShapeDtypeStruct((2, N, H), jnp.float32),
    )(h, w_stack)


def _gather_body(t_hbm, ridx_hbm, cidx_hbm, g_hbm,
                 ridx_v, cidx_v, buf_a, buf_b, sem_a, sem_b):
    wid = lax.axis_index("s") * NC + lax.axis_index("c")

    def chunk_body(k, carry):
        chunk = k * NW + wid

        @pl.when(chunk < NCHUNK)
        def _():
            base = chunk * CHUNK
            pltpu.sync_copy(ridx_hbm.at[pl.ds(base, CHUNK)], ridx_v)
            pltpu.sync_copy(cidx_hbm.at[pl.ds(base, CHUNK)], cidx_v)
            cp_a = pltpu.async_copy(t_hbm.at[ridx_v], buf_a, sem_a)
            cp_b = pltpu.async_copy(t_hbm.at[cidx_v], buf_b, sem_b)
            cp_a.wait()
            cp_b.wait()

            def add_body(e, c2):
                for j in range(H // L):
                    s = pl.ds(j * L, L)
                    buf_a[e, s] = buf_a[e, s] + buf_b[e, s]
                return c2

            lax.fori_loop(0, CHUNK, add_body, 0)
            pltpu.sync_copy(buf_a, g_hbm.at[pl.ds(base, CHUNK), :])

        return carry

    lax.fori_loop(0, KMAX, chunk_body, 0)


def _gather(t_table, ridx, cidx):
    mesh = plsc.VectorSubcoreMesh(
        core_axis_name="c", subcore_axis_name="s",
        num_cores=NC, num_subcores=NS)
    f = pl.kernel(
        _gather_body,
        out_type=jax.ShapeDtypeStruct((E, H), jnp.float32),
        mesh=mesh,
        scratch_types=[
            pltpu.VMEM((CHUNK,), jnp.int32),
            pltpu.VMEM((CHUNK,), jnp.int32),
            pltpu.VMEM((CHUNK, H), jnp.float32),
            pltpu.VMEM((CHUNK, H), jnp.float32),
            pltpu.SemaphoreType.DMA,
            pltpu.SemaphoreType.DMA,
        ],
    )
    return f(t_table, ridx, cidx)


def _mlp_body(g_ref, ea_ref, cd_ref, w1c_ref, b1_ref, w2_ref, b2_ref, w3_ref,
              out_ref):
    z = (g_ref[...]
         + jnp.dot(ea_ref[...], w1c_ref[...], preferred_element_type=jnp.float32)
         + b1_ref[...])
    x1 = z * jax.nn.sigmoid(z)
    y = jnp.dot(x1.astype(jnp.bfloat16), w2_ref[...],
                preferred_element_type=jnp.float32) + b2_ref[...]
    x2 = y * jax.nn.sigmoid(y)
    m = jnp.sum(x2 * w3_ref[...], axis=1, keepdims=True)
    out_ref[...] = cd_ref[...] * (jnp.tanh(m) * CRANGE)


def _mlp(g, ea8, cd4, w1c8, b1r, w2t, b2r, w3r):
    grid = (E // BE,)
    return pl.pallas_call(
        _mlp_body,
        grid=grid,
        in_specs=[
            pl.BlockSpec((BE, H), lambda i: (i, 0)),
            pl.BlockSpec((BE, 8), lambda i: (i, 0)),
            pl.BlockSpec((BE, 4), lambda i: (i, 0)),
            pl.BlockSpec((8, H), lambda i: (0, 0)),
            pl.BlockSpec((1, H), lambda i: (0, 0)),
            pl.BlockSpec((H, H), lambda i: (0, 0)),
            pl.BlockSpec((1, H), lambda i: (0, 0)),
            pl.BlockSpec((1, H), lambda i: (0, 0)),
        ],
        out_specs=pl.BlockSpec((BE, 4), lambda i: (i, 0)),
        out_shape=jax.ShapeDtypeStruct((E, 4), jnp.float32),
    )(g, ea8, cd4, w1c8, b1r, w2t, b2r, w3r)


ZSEG = 1000


def _scatter_body(t_hbm, ridx_hbm, zeros_hbm, part_hbm, ridx_v, tval_v, shared):
    cid = lax.axis_index("c")
    sid = lax.axis_index("s")
    wid = sid * NC + cid

    @pl.when(sid < 10)
    def _():
        pltpu.sync_copy(zeros_hbm.at[pl.ds(sid * ZSEG, ZSEG), :],
                        shared.at[pl.ds(sid * ZSEG, ZSEG), :])

    plsc.subcore_barrier()

    def chunk_body(k, carry):
        chunk = k * NW + wid

        @pl.when(chunk < NCHUNK)
        def _():
            base = chunk * CHUNK
            pltpu.sync_copy(ridx_hbm.at[pl.ds(base, CHUNK)], ridx_v)
            pltpu.sync_copy(t_hbm.at[pl.ds(base, CHUNK), :], tval_v)
            pltpu.sync_copy(tval_v, shared.at[ridx_v], add=True)

        return carry

    lax.fori_loop(0, KMAX, chunk_body, 0)
    plsc.subcore_barrier()

    @pl.when(sid < 10)
    def _():
        pltpu.sync_copy(shared.at[pl.ds(sid * ZSEG, ZSEG), :],
                        part_hbm.at[cid, pl.ds(sid * ZSEG, ZSEG), :])


def _scatter(t4, ridx, zeros2d):
    mesh = plsc.VectorSubcoreMesh(
        core_axis_name="c", subcore_axis_name="s",
        num_cores=NC, num_subcores=NS)
    f = pl.kernel(
        _scatter_body,
        out_type=jax.ShapeDtypeStruct((NC, N, 4), jnp.float32),
        mesh=mesh,
        scratch_types=[
            pltpu.VMEM((CHUNK,), jnp.int32),
            pltpu.VMEM((CHUNK, 4), jnp.float32),
            pltpu.VMEM_SHARED((N, 4), jnp.float32),
        ],
        compiler_params=pltpu.CompilerParams(needs_layout_passes=False),
    )
    return f(t4, ridx, zeros2d)


def _final_body(p_ref, c_ref, out_ref):
    s = jnp.sum(p_ref[...], axis=0, keepdims=True)
    out_ref[...] = c_ref[...] + s * NORM_INV


def _final(part, coordp):
    return pl.pallas_call(
        _final_body,
        in_specs=[
            pl.BlockSpec((NC, N4), lambda: (0, 0)),
            pl.BlockSpec((1, N4), lambda: (0, 0)),
        ],
        out_specs=pl.BlockSpec((1, N4), lambda: (0, 0)),
        out_shape=jax.ShapeDtypeStruct((1, N4), jnp.float32),
    )(part, coordp)


def kernel(h, coord, edge_index, coord_diff, edge_attr, w1, b1, w2, b2, w3):
    row = edge_index[0].astype(jnp.int32)
    col = edge_index[1].astype(jnp.int32)

    w_stack = jnp.stack([w1[:, :H].T, w1[:, H:2 * H].T])
    t_table = _project(h, w_stack).reshape(2 * N, H)

    g = _gather(t_table, row, col + N)

    w1c8 = jnp.pad(w1[:, 2 * H:].T, ((0, 4), (0, 0)))
    ea8 = jnp.pad(edge_attr, ((0, 0), (0, 4)))
    cd4 = jnp.pad(coord_diff, ((0, 0), (0, 1)))
    t4 = _mlp(g, ea8, cd4, w1c8,
              b1.reshape(1, H), w2.T.astype(jnp.bfloat16),
              b2.reshape(1, H), w3)

    zeros2d = jnp.zeros((N, 4), jnp.float32)
    part = _scatter(t4, row, zeros2d).reshape(NC, N4)

    coordp = jnp.pad(coord, ((0, 0), (0, 1))).reshape(1, N4)
    out = _final(part, coordp)
    return out.reshape(-1, 4)[:N, :3]
